# Initial kernel scaffold; baseline (speedup 1.0000x reference)
import jax
import jax.numpy as jnp
from jax.experimental import pallas as pl


def kernel(x, units):
    raise NotImplementedError("write your pallas kernel here")


# speedup vs baseline: 10.7928x; 10.7928x over previous
import math

import jax
import jax.numpy as jnp
from jax.experimental import pallas as pl
from jax.experimental.pallas import tpu as pltpu

_G = 32
_MAX_T = 10000
_STEP_T = 1
_T = min(1 + _STEP_T, _MAX_T)
_DENO = math.log(_G) / (0.9 * _MAX_T)
_KSIZE = _G * math.exp(-_T * _DENO)
_SIGMA = 0.3 * ((_KSIZE - 1) * 0.5 - 1) + 0.8
_TWO_SIGMA_SQ = 2.0 * _SIGMA**2
_TEMP = math.exp(-(_T * 2) / _MAX_T)


def _som_body(x_ref, u_ref, bmu_ref, loss_ref):
    b = x_ref.shape[0]
    n = u_ref.shape[0]
    d = u_ref.shape[1]
    xc = x_ref[:] - 0.5
    uc = u_ref[:] - 0.5
    dn = (((1,), (1,)), ((), ()))
    dot = jax.lax.dot_general(
        xc, uc, dn, preferred_element_type=jnp.float32,
        precision=jax.lax.Precision.HIGHEST)
    xn = jnp.sum(xc * xc, axis=1, keepdims=True)
    un = jax.lax.dot_general(
        jnp.ones((1, d), jnp.float32), uc * uc, dn,
        preferred_element_type=jnp.float32,
        precision=jax.lax.Precision.HIGHEST)
    dist = (xn - 2.0 * dot) + un

    col = jax.lax.broadcasted_iota(jnp.int32, (b, n), 1)
    dmin = jnp.min(dist, axis=1, keepdims=True)
    bmu = jnp.min(jnp.where(dist == dmin, col, n), axis=1,
                  keepdims=True)

    by = bmu // _G
    bx = bmu % _G
    uy = col // _G
    ux = col % _G
    pd = ((uy - by) ** 2 + (ux - bx) ** 2).astype(jnp.float32)
    gauss = jnp.exp(-pd / _TWO_SIGMA_SQ)
    gauss = jnp.where(gauss < 0.001, 0.0, gauss)
    wsum = jnp.sum(gauss * gauss * dist)
    loss_ref[0, 0] = wsum * (_TEMP * _TEMP / (b * n * d))
    bmu_ref[:] = bmu


def kernel(x, units):
    bsz = x.shape[0]
    d = units.shape[-1]
    n = units.shape[0] * units.shape[1]
    x2 = x.reshape(bsz, d)
    u2 = units.reshape(n, d)
    bmu, loss = pl.pallas_call(
        _som_body,
        out_shape=(
            jax.ShapeDtypeStruct((bsz, 1), jnp.int32),
            jax.ShapeDtypeStruct((1, 1), jnp.float32),
        ),
        out_specs=(
            pl.BlockSpec(memory_space=pltpu.VMEM),
            pl.BlockSpec(memory_space=pltpu.SMEM),
        ),
    )(x2, u2)
    return bmu, loss.reshape(())
